# baseline (speedup 1.0000x reference)
import functools

import jax
import jax.numpy as jnp
from jax import lax
from jax.experimental import pallas as pl
from jax.experimental.pallas import tpu as pltpu
from jax.experimental.pallas import tpu_sc as plsc

_SHIFT = 8192
_B, _S, _C, _T = 8, 4, 2, 220500
_L = _T - _SHIFT
_NW = 32
_LPAD = 212352
_CKO = 16256
_CKI = _CKO + 128
_NVREG = _CKO // 16
_UNROLL = 8

_STARTS = [_k * _CKO for _k in range(_LPAD // _CKO)] + [_LPAD - _CKO]


def _body(wav_hbm, off_hbm, out_hbm, off_v, ib0, ib1, ob0, ob1,
          si0, si1, so0, so1):
    cid = lax.axis_index("c")
    sid = lax.axis_index("s")
    wid = sid * 2 + cid

    pltpu.sync_copy(off_hbm, off_v)
    v_lo = off_v[pl.ds(0, 16)]
    v_hi = off_v[pl.ds(16, 16)]
    v = jnp.where(wid < 16, v_lo, v_hi)
    lanes = lax.iota(jnp.int32, 16)
    off = jnp.sum(jnp.where(lanes == wid % 16, v, 0))

    r = off % 128
    off_al = off - r

    ibufs = (ib0, ib1)
    obufs = (ob0, ob1)
    sem_in = (si0, si1)
    sem_out = (so0, so1)
    n = len(_STARTS)
    cp_in = []
    cp_out = []
    for t, t0 in enumerate(_STARTS):
        b = t % 2
        src = pl.multiple_of(off_al + t0, 128)
        dst = pl.multiple_of(off * 0 + t0, 128)
        cp_in.append(pltpu.make_async_copy(
            wav_hbm.at[wid, :, pl.ds(src, _CKI)], ibufs[b], sem_in[b]))
        cp_out.append(pltpu.make_async_copy(
            obufs[b], out_hbm.at[wid, :, pl.ds(dst, _CKO)], sem_out[b]))

    def shift_chunk(ib, ob):
        for ch in range(2):
            ch_idx = jnp.full((16,), ch, jnp.int32)

            @plsc.parallel_loop(0, _NVREG, unroll=_UNROLL)
            def _(i):
                o = pl.multiple_of(i * 16, 16)
                ob[ch, pl.ds(o, 16)] = plsc.load_gather(
                    ib, [ch_idx, lanes + (r + o)])

    cp_in[0].start()
    for t in range(n):
        cp_in[t].wait()
        if t + 1 < n:
            cp_in[t + 1].start()
        if t >= 2:
            cp_out[t - 2].wait()
        shift_chunk(ibufs[t % 2], obufs[t % 2])
        cp_out[t].start()

    cp_out[n - 2].wait()
    cp_out[n - 1].wait()


@jax.jit
def kernel(wav, offsets):
    wav3 = wav.reshape(_NW, _C, _T)
    off1 = offsets.reshape(_NW).astype(jnp.int32)
    mesh = plsc.VectorSubcoreMesh(core_axis_name="c", subcore_axis_name="s")
    run = functools.partial(
        pl.kernel,
        mesh=mesh,
        compiler_params=pltpu.CompilerParams(
            needs_layout_passes=False, disable_bounds_checks=True),
        out_type=jax.ShapeDtypeStruct((_NW, _C, _L), jnp.float32),
        scratch_types=[
            pltpu.VMEM((_NW,), jnp.int32),
            pltpu.VMEM((_C, _CKI), jnp.float32),
            pltpu.VMEM((_C, _CKI), jnp.float32),
            pltpu.VMEM((_C, _CKO), jnp.float32),
            pltpu.VMEM((_C, _CKO), jnp.float32),
            pltpu.SemaphoreType.DMA,
            pltpu.SemaphoreType.DMA,
            pltpu.SemaphoreType.DMA,
            pltpu.SemaphoreType.DMA,
        ],
    )(_body)
    out = run(wav3, off1)
    return out.reshape(_B, _S, _C, _L)


# speedup vs baseline: 1.0464x; 1.0464x over previous
import functools

import jax
import jax.numpy as jnp
from jax import lax
from jax.experimental import pallas as pl
from jax.experimental.pallas import tpu as pltpu
from jax.experimental.pallas import tpu_sc as plsc

_SHIFT = 8192
_B, _S, _C, _T = 8, 4, 2, 220500
_L = _T - _SHIFT
_NW = 32
_LPAD = 212352
_CKO = 16256
_CKI = _CKO + 128
_NVREG = _CKO // 16
_UNROLL = 8

_STARTS = [(_k * _CKO, _CKO) for _k in range(_LPAD // _CKO)]
_STARTS.append((_STARTS[-1][0] + _CKO, _LPAD - len(_STARTS) * _CKO))


def _body(wav_hbm, off_hbm, out_hbm, off_v, ib0, ib1, ob0, ob1,
          si0, si1, so0, so1):
    cid = lax.axis_index("c")
    sid = lax.axis_index("s")
    wid = sid * 2 + cid

    pltpu.sync_copy(off_hbm, off_v)
    v_lo = off_v[pl.ds(0, 16)]
    v_hi = off_v[pl.ds(16, 16)]
    v = jnp.where(wid < 16, v_lo, v_hi)
    lanes = lax.iota(jnp.int32, 16)
    off = jnp.sum(jnp.where(lanes == wid % 16, v, 0))

    r = off % 128
    off_al = off - r

    ibufs = (ib0, ib1)
    obufs = (ob0, ob1)
    sem_in = (si0, si1)
    sem_out = (so0, so1)
    n = len(_STARTS)
    cp_in = []
    cp_out = []
    for t, (t0, sz) in enumerate(_STARTS):
        b = t % 2
        src = pl.multiple_of(off_al + t0, 128)
        dst = pl.multiple_of(off * 0 + t0, 128)
        cp_in.append(pltpu.make_async_copy(
            wav_hbm.at[wid, :, pl.ds(src, sz + 128)],
            ibufs[b].at[:, pl.ds(0, sz + 128)], sem_in[b]))
        cp_out.append(pltpu.make_async_copy(
            obufs[b].at[:, pl.ds(0, sz)],
            out_hbm.at[wid, :, pl.ds(dst, sz)], sem_out[b]))

    def shift_chunk(ib, ob, sz):
        for ch in range(2):
            ch_idx = jnp.full((16,), ch, jnp.int32)

            @plsc.parallel_loop(0, sz // 16, unroll=_UNROLL)
            def _(i):
                o = pl.multiple_of(i * 16, 16)
                ob[ch, pl.ds(o, 16)] = plsc.load_gather(
                    ib, [ch_idx, lanes + (r + o)])

    cp_in[0].start()
    for t in range(n):
        cp_in[t].wait()
        if t + 1 < n:
            cp_in[t + 1].start()
        if t >= 2:
            cp_out[t - 2].wait()
        shift_chunk(ibufs[t % 2], obufs[t % 2], _STARTS[t][1])
        cp_out[t].start()

    cp_out[n - 2].wait()
    cp_out[n - 1].wait()


@jax.jit
def kernel(wav, offsets):
    wav3 = wav.reshape(_NW, _C, _T)
    off1 = offsets.reshape(_NW).astype(jnp.int32)
    mesh = plsc.VectorSubcoreMesh(core_axis_name="c", subcore_axis_name="s")
    run = functools.partial(
        pl.kernel,
        mesh=mesh,
        compiler_params=pltpu.CompilerParams(
            needs_layout_passes=False, disable_bounds_checks=True),
        out_type=jax.ShapeDtypeStruct((_NW, _C, _L), jnp.float32),
        scratch_types=[
            pltpu.VMEM((_NW,), jnp.int32),
            pltpu.VMEM((_C, _CKI), jnp.float32),
            pltpu.VMEM((_C, _CKI), jnp.float32),
            pltpu.VMEM((_C, _CKO), jnp.float32),
            pltpu.VMEM((_C, _CKO), jnp.float32),
            pltpu.SemaphoreType.DMA,
            pltpu.SemaphoreType.DMA,
            pltpu.SemaphoreType.DMA,
            pltpu.SemaphoreType.DMA,
        ],
    )(_body)
    out = run(wav3, off1)
    return out.reshape(_B, _S, _C, _L)
